# baseline (speedup 1.0000x reference)
import functools

import jax
import jax.numpy as jnp
from jax import lax
from jax.experimental import pallas as pl
from jax.experimental.pallas import tpu as pltpu

try:
    from jax.experimental.pallas import tpu_sc as plsc
    _HAS_SC = True
except ImportError:
    _HAS_SC = False

_D = 128
_N = 200
_BB = 32


def _sc_gather(idx, table):
    b = idx.shape[0]
    info = plsc.get_sparse_core_info()
    nw = info.num_cores * info.num_subcores
    b_per_w = b // nw
    mesh = plsc.VectorSubcoreMesh(core_axis_name="c", subcore_axis_name="s")

    @functools.partial(
        pl.kernel,
        mesh=mesh,
        out_type=jax.ShapeDtypeStruct((b, table.shape[1]), jnp.float32),
        scratch_types=[
            pltpu.VMEM((b_per_w,), jnp.int32),
            pltpu.VMEM((b_per_w, table.shape[1]), jnp.float32),
            pltpu.SemaphoreType.DMA,
        ],
    )
    def k(idx_hbm, table_hbm, out_hbm, idx_v, rows_v, sem):
        wid = lax.axis_index("s") * info.num_cores + lax.axis_index("c")
        base = wid * b_per_w
        pltpu.sync_copy(idx_hbm.at[pl.ds(base, b_per_w)], idx_v)
        pltpu.async_copy(table_hbm.at[idx_v], rows_v, sem).wait()
        pltpu.sync_copy(rows_v, out_hbm.at[pl.ds(base, b_per_w)])

    return k(idx, table)


_NDMA = 16


def _bcast_body(x_ref, o_ref, sem):
    def cp(n):
        return pltpu.make_async_copy(x_ref, o_ref.at[:, n, :], sem)

    for n in range(_N):
        cp(n).start()
        if n >= _NDMA:
            cp(n - _NDMA).wait()
    for n in range(_N - _NDMA, _N):
        cp(n).wait()


def _tc_broadcast(x):
    b = x.shape[0]
    return pl.pallas_call(
        _bcast_body,
        in_specs=[pl.BlockSpec(memory_space=pltpu.VMEM)],
        out_specs=pl.BlockSpec(memory_space=pl.ANY),
        out_shape=jax.ShapeDtypeStruct((b, _N, _D), jnp.float32),
        scratch_shapes=[pltpu.SemaphoreType.DMA],
    )(x)


def kernel(attr_idx, embedding):
    idx = attr_idx.astype(jnp.int32)
    x = _sc_gather(idx, embedding)
    return _tc_broadcast(x)


# speedup vs baseline: 1.0016x; 1.0016x over previous
import functools

import jax
import jax.numpy as jnp
from jax import lax
from jax.experimental import pallas as pl
from jax.experimental.pallas import tpu as pltpu

try:
    from jax.experimental.pallas import tpu_sc as plsc
    _HAS_SC = True
except ImportError:
    _HAS_SC = False

_D = 128
_N = 200
_BB = 32


def _sc_gather(idx, table):
    b = idx.shape[0]
    info = plsc.get_sparse_core_info()
    ncores = 1
    nw = ncores * info.num_subcores
    b_per_w = b // nw
    mesh = plsc.VectorSubcoreMesh(
        core_axis_name="c", subcore_axis_name="s", num_cores=ncores
    )

    @functools.partial(
        pl.kernel,
        mesh=mesh,
        out_type=jax.ShapeDtypeStruct((b, table.shape[1]), jnp.float32),
        scratch_types=[
            pltpu.VMEM((b_per_w,), jnp.int32),
            pltpu.VMEM((b_per_w, table.shape[1]), jnp.float32),
            pltpu.SemaphoreType.DMA,
        ],
    )
    def k(idx_hbm, table_hbm, out_hbm, idx_v, rows_v, sem):
        wid = lax.axis_index("s") * ncores + lax.axis_index("c")
        base = wid * b_per_w
        pltpu.sync_copy(idx_hbm.at[pl.ds(base, b_per_w)], idx_v)
        pltpu.async_copy(table_hbm.at[idx_v], rows_v, sem).wait()
        pltpu.sync_copy(rows_v, out_hbm.at[pl.ds(base, b_per_w)])

    return k(idx, table)


_NDMA = 16


def _bcast_body(x_ref, o_ref, sem):
    def cp(n):
        return pltpu.make_async_copy(x_ref, o_ref.at[:, n, :], sem)

    for n in range(_N):
        cp(n).start()
        if n >= _NDMA:
            cp(n - _NDMA).wait()
    for n in range(_N - _NDMA, _N):
        cp(n).wait()


def _tc_broadcast(x):
    b = x.shape[0]
    return pl.pallas_call(
        _bcast_body,
        in_specs=[pl.BlockSpec(memory_space=pltpu.VMEM)],
        out_specs=pl.BlockSpec(memory_space=pl.ANY),
        out_shape=jax.ShapeDtypeStruct((b, _N, _D), jnp.float32),
        scratch_shapes=[pltpu.SemaphoreType.DMA],
    )(x)


def kernel(attr_idx, embedding):
    idx = attr_idx.astype(jnp.int32)
    x = _sc_gather(idx, embedding)
    return _tc_broadcast(x)
